# Initial kernel scaffold; baseline (speedup 1.0000x reference)
import jax
import jax.numpy as jnp
from jax.experimental import pallas as pl


def kernel(x_in, y_target, W_enc, W_dec, b_dec):
    raise NotImplementedError("write your pallas kernel here")


# speedup vs baseline: 4.8855x; 4.8855x over previous
import jax
import jax.numpy as jnp
from jax import lax
from jax.experimental import pallas as pl
from jax.experimental.pallas import tpu as pltpu

L1_COEFF = 1e-3
TOPK = 32


def _encode_call(x_in, bdec2, W_enc, tile_r, nj):
    n_tok, d_in = x_in.shape
    d_dict = W_enc.shape[1]
    ni = n_tok // tile_r
    tile_d = d_dict // nj

    def body(x_ref, bdec_ref, we_ref, out_ref, xn_s, acts_s):
        j = pl.program_id(1)

        @pl.when(j == 0)
        def _():
            x = x_ref[...]
            mu = jnp.mean(x, axis=1, keepdims=True)
            xc = x - mu
            var = jnp.sum(xc * xc, axis=1, keepdims=True) * (1.0 / (d_in - 1))
            sd = jnp.sqrt(var)
            xn_s[...] = xc / (sd + 1e-5) - bdec_ref[...]

        a = jnp.dot(xn_s[...], we_ref[...], preferred_element_type=jnp.float32)
        acts_s[:, pl.ds(j * tile_d, tile_d)] = jnp.maximum(a, 0.0)

        @pl.when(j == nj - 1)
        def _():
            acts = acts_s[...]

            def it(_, t):
                return jnp.max(jnp.where(acts < t, acts, -jnp.inf),
                               axis=1, keepdims=True)

            t32 = lax.fori_loop(0, TOPK, it,
                                jnp.full((tile_r, 1), jnp.inf, jnp.float32))
            mask = (acts >= t32) & (acts > 0.0)
            out_ref[...] = jnp.where(mask, acts, 0.0)

    return pl.pallas_call(
        body,
        grid=(ni, nj),
        in_specs=[
            pl.BlockSpec((tile_r, d_in), lambda i, j: (i, 0)),
            pl.BlockSpec((1, d_in), lambda i, j: (0, 0)),
            pl.BlockSpec((d_in, tile_d), lambda i, j: (0, j)),
        ],
        out_specs=pl.BlockSpec((tile_r, d_dict), lambda i, j: (i, 0)),
        out_shape=jax.ShapeDtypeStruct((n_tok, d_dict), jnp.float32),
        scratch_shapes=[
            pltpu.VMEM((tile_r, d_in), jnp.float32),
            pltpu.VMEM((tile_r, d_dict), jnp.float32),
        ],
        compiler_params=pltpu.CompilerParams(
            dimension_semantics=("arbitrary", "arbitrary")),
    )(x_in, bdec2, W_enc)


def _decode_call(acts_sparse, W_dec, y_target, bdec2, tile_r, nk):
    n_tok, d_dict = acts_sparse.shape
    d_in = W_dec.shape[1]
    ni = n_tok // tile_r
    tile_k = d_dict // nk

    def body(asp_ref, wd_ref, y_ref, bdec_ref, ypo_ref, sums_ref, yp_s, sm):
        i = pl.program_id(0)
        kk = pl.program_id(1)

        @pl.when((i == 0) & (kk == 0))
        def _():
            sm[0] = 0.0
            sm[1] = 0.0
            sm[2] = 0.0

        asp = asp_ref[...]
        sm[0] = sm[0] + jnp.sum(asp)
        sm[1] = sm[1] + jnp.sum((asp > 0.0).astype(jnp.float32))

        part = jnp.dot(asp, wd_ref[...], preferred_element_type=jnp.float32)

        @pl.when(kk == 0)
        def _():
            yp_s[...] = part

        @pl.when(kk > 0)
        def _():
            yp_s[...] = yp_s[...] + part

        @pl.when(kk == nk - 1)
        def _():
            yp = yp_s[...] + bdec_ref[...]
            y = y_ref[...]
            mu = jnp.mean(y, axis=1, keepdims=True)
            yc = y - mu
            var = jnp.sum(yc * yc, axis=1, keepdims=True) * (1.0 / (d_in - 1))
            sd = jnp.sqrt(var)
            yn = yc / (sd + 1e-5)
            sm[2] = sm[2] + jnp.sum((yp - yn) ** 2)
            ypo_ref[...] = yp * sd + mu

        @pl.when((i == ni - 1) & (kk == nk - 1))
        def _():
            r = lax.broadcasted_iota(jnp.int32, (8, 128), 0)
            c = lax.broadcasted_iota(jnp.int32, (8, 128), 1)
            z = jnp.zeros((8, 128), jnp.float32)
            vals = jnp.where((r == 0) & (c == 0), sm[0], z)
            vals = jnp.where((r == 1) & (c == 0), sm[1], vals)
            vals = jnp.where((r == 2) & (c == 0), sm[2], vals)
            sums_ref[...] = vals

    return pl.pallas_call(
        body,
        grid=(ni, nk),
        in_specs=[
            pl.BlockSpec((tile_r, tile_k), lambda i, k: (i, k)),
            pl.BlockSpec((tile_k, d_in), lambda i, k: (k, 0)),
            pl.BlockSpec((tile_r, d_in), lambda i, k: (i, 0)),
            pl.BlockSpec((1, d_in), lambda i, k: (0, 0)),
        ],
        out_specs=[
            pl.BlockSpec((tile_r, d_in), lambda i, k: (i, 0)),
            pl.BlockSpec((8, 128), lambda i, k: (0, 0)),
        ],
        out_shape=[
            jax.ShapeDtypeStruct((n_tok, d_in), jnp.float32),
            jax.ShapeDtypeStruct((8, 128), jnp.float32),
        ],
        scratch_shapes=[
            pltpu.VMEM((tile_r, d_in), jnp.float32),
            pltpu.SMEM((4,), jnp.float32),
        ],
        compiler_params=pltpu.CompilerParams(
            dimension_semantics=("arbitrary", "arbitrary")),
    )(acts_sparse, W_dec, y_target, bdec2)


def kernel(x_in, y_target, W_enc, W_dec, b_dec):
    n_tok, d_in = x_in.shape
    d_dict = W_enc.shape[1]
    tile_r = 128 if n_tok % 128 == 0 else n_tok
    nj = 8 if d_dict % (8 * 128) == 0 else 1
    bdec2 = b_dec.reshape(1, d_in)

    acts_sparse = _encode_call(x_in, bdec2, W_enc, tile_r, nj)
    y_pred_out, sums = _decode_call(acts_sparse, W_dec, y_target, bdec2,
                                    tile_r, nj)

    l1_norm = sums[0, 0] / n_tok
    l0_norm = sums[1, 0] / n_tok
    l2_loss = sums[2, 0] / (n_tok * d_in)
    l1_loss = L1_COEFF * l1_norm
    aux_loss = jnp.array(0.0, jnp.float32)
    loss = l2_loss + l1_loss + aux_loss
    num_dead_features = jnp.array(0, jnp.int32)
    return (y_pred_out, acts_sparse, loss, l1_loss, l2_loss,
            l0_norm, l1_norm, aux_loss, num_dead_features)
